# Initial kernel scaffold; baseline (speedup 1.0000x reference)
import functools

import jax
import jax.numpy as jnp
from jax import lax
from jax.experimental import pallas as pl
from jax.experimental.pallas import tpu as pltpu
from jax.experimental.pallas import tpu_sc as plsc

NC = 2
NS = 16
NW = NC * NS
CHUNK = 128


def _sc_gather(sense_emb, v_emb, sidx2, vidx2, b, n_v):
    d = sense_emb.shape[1]
    s_chunks = b // CHUNK
    v_chunks = n_v // CHUNK
    s_per_w = s_chunks // NW
    v_per_w = v_chunks // NW
    mesh = plsc.VectorSubcoreMesh(core_axis_name="c", subcore_axis_name="s")

    @functools.partial(
        pl.kernel,
        mesh=mesh,
        out_type=[
            jax.ShapeDtypeStruct((b, d), jnp.float32),
            jax.ShapeDtypeStruct((n_v, d), jnp.float32),
        ],
        scratch_types=[
            pltpu.VMEM((s_per_w, CHUNK), jnp.int32),
            pltpu.VMEM((v_per_w, CHUNK), jnp.int32),
            pltpu.VMEM((CHUNK, d), jnp.float32),
            pltpu.VMEM((CHUNK, d), jnp.float32),
            pltpu.SemaphoreType.DMA,
            pltpu.SemaphoreType.DMA,
        ],
    )
    def k(sense_hbm, vemb_hbm, sidx_hbm, vidx_hbm, sout_hbm, vout_hbm,
          sidx_v, vidx_v, rows_a, rows_b, sem_a, sem_b):
        wid = lax.axis_index("s") * NC + lax.axis_index("c")

        pltpu.sync_copy(sidx_hbm.at[pl.ds(wid * s_per_w, s_per_w)], sidx_v)
        pltpu.sync_copy(vidx_hbm.at[pl.ds(wid * v_per_w, v_per_w)], vidx_v)

        @pl.loop(0, s_per_w, step=2)
        def _(j):
            pltpu.async_copy(sense_hbm.at[sidx_v.at[j]], rows_a, sem_a).wait()
            cp_a = pltpu.make_async_copy(
                rows_a, sout_hbm.at[pl.ds((wid * s_per_w + j) * CHUNK, CHUNK)], sem_a)
            cp_a.start()
            pltpu.async_copy(sense_hbm.at[sidx_v.at[j + 1]], rows_b, sem_b).wait()
            cp_b = pltpu.make_async_copy(
                rows_b, sout_hbm.at[pl.ds((wid * s_per_w + j + 1) * CHUNK, CHUNK)], sem_b)
            cp_b.start()
            cp_a.wait()
            cp_b.wait()

        @pl.loop(0, v_per_w, step=2)
        def _(j):
            pltpu.async_copy(vemb_hbm.at[vidx_v.at[j]], rows_a, sem_a).wait()
            cp_a = pltpu.make_async_copy(
                rows_a, vout_hbm.at[pl.ds((wid * v_per_w + j) * CHUNK, CHUNK)], sem_a)
            cp_a.start()
            pltpu.async_copy(vemb_hbm.at[vidx_v.at[j + 1]], rows_b, sem_b).wait()
            cp_b = pltpu.make_async_copy(
                rows_b, vout_hbm.at[pl.ds((wid * v_per_w + j + 1) * CHUNK, CHUNK)], sem_b)
            cp_b.start()
            cp_a.wait()
            cp_b.wait()

    return k(sense_emb, v_emb, sidx2, vidx2)


def _tc_loss(sense_rows, v3, b, n_neg, d, blk):
    steps = b // blk

    def body(s_ref, v_ref, o_ref):
        i = pl.program_id(0)
        es = s_ref[...]
        total = jnp.sum(jax.nn.log_sigmoid(jnp.sum(es * v_ref[0], axis=1)))
        for n in range(n_neg):
            q = jnp.sum(es * v_ref[n + 1], axis=1)
            total += jnp.sum(jax.nn.log_sigmoid(-q))

        @pl.when(i == 0)
        def _():
            o_ref[0, 0] = 0.0

        o_ref[0, 0] += -total

    return pl.pallas_call(
        body,
        grid=(steps,),
        in_specs=[
            pl.BlockSpec((blk, d), lambda i: (i, 0)),
            pl.BlockSpec((1 + n_neg, blk, d), lambda i: (0, i, 0)),
        ],
        out_specs=pl.BlockSpec((1, 1), lambda i: (0, 0)),
        out_shape=jax.ShapeDtypeStruct((1, 1), jnp.float32),
    )(sense_rows, v3)


def kernel(pos_u, pos_v, neg_v, rightsense, v_emb, sense_emb):
    b = pos_u.shape[0]
    n_neg = neg_v.shape[1]
    d = v_emb.shape[1]
    k_senses = sense_emb.shape[0] // v_emb.shape[0]

    rs = jnp.asarray(rightsense, dtype=jnp.int32)
    sense_idx = pos_u.astype(jnp.int32) * jnp.int32(k_senses) + rs
    v_idx = jnp.concatenate([pos_v[None, :], neg_v.T], axis=0).reshape(-1)
    n_v = (1 + n_neg) * b

    sidx2 = sense_idx.reshape(b // CHUNK, CHUNK)
    vidx2 = v_idx.reshape(n_v // CHUNK, CHUNK)

    sense_rows, v_rows = _sc_gather(sense_emb, v_emb, sidx2, vidx2, b, n_v)
    v3 = v_rows.reshape(1 + n_neg, b, d)
    out = _tc_loss(sense_rows, v3, b, n_neg, d, blk=2048)
    return out.reshape(())


# speedup vs baseline: 1.1931x; 1.1931x over previous
import functools

import jax
import jax.numpy as jnp
from jax import lax
from jax.experimental import pallas as pl
from jax.experimental.pallas import tpu as pltpu
from jax.experimental.pallas import tpu_sc as plsc

NC = 2
NS = 16
NW = NC * NS
CHUNK = 128


def _sc_gather(sense_emb, v_emb, sidx2, vidx2, b, n_v):
    d = sense_emb.shape[1]
    s_chunks = b // CHUNK
    v_chunks = n_v // CHUNK
    s_per_w = s_chunks // NW
    v_per_w = v_chunks // NW
    mesh = plsc.VectorSubcoreMesh(core_axis_name="c", subcore_axis_name="s")

    @functools.partial(
        pl.kernel,
        mesh=mesh,
        compiler_params=pltpu.CompilerParams(use_tc_tiling_on_sc=False),
        out_type=[
            jax.ShapeDtypeStruct((b, d), jnp.float32),
            jax.ShapeDtypeStruct((n_v, d), jnp.float32),
        ],
        scratch_types=[
            pltpu.VMEM((s_per_w, CHUNK), jnp.int32),
            pltpu.VMEM((v_per_w, CHUNK), jnp.int32),
            pltpu.VMEM((CHUNK, d), jnp.float32),
            pltpu.VMEM((CHUNK, d), jnp.float32),
            pltpu.SemaphoreType.DMA,
            pltpu.SemaphoreType.DMA,
        ],
    )
    def k(sense_hbm, vemb_hbm, sidx_hbm, vidx_hbm, sout_hbm, vout_hbm,
          sidx_v, vidx_v, rows_a, rows_b, sem_a, sem_b):
        wid = lax.axis_index("s") * NC + lax.axis_index("c")

        pltpu.sync_copy(sidx_hbm.at[pl.ds(wid * s_per_w, s_per_w)], sidx_v)
        pltpu.sync_copy(vidx_hbm.at[pl.ds(wid * v_per_w, v_per_w)], vidx_v)

        @pl.loop(0, s_per_w, step=2)
        def _(j):
            pltpu.async_copy(sense_hbm.at[sidx_v.at[j]], rows_a, sem_a).wait()
            cp_a = pltpu.make_async_copy(
                rows_a, sout_hbm.at[pl.ds((wid * s_per_w + j) * CHUNK, CHUNK)], sem_a)
            cp_a.start()
            pltpu.async_copy(sense_hbm.at[sidx_v.at[j + 1]], rows_b, sem_b).wait()
            cp_b = pltpu.make_async_copy(
                rows_b, sout_hbm.at[pl.ds((wid * s_per_w + j + 1) * CHUNK, CHUNK)], sem_b)
            cp_b.start()
            cp_a.wait()
            cp_b.wait()

        @pl.loop(0, v_per_w, step=2)
        def _(j):
            pltpu.async_copy(vemb_hbm.at[vidx_v.at[j]], rows_a, sem_a).wait()
            cp_a = pltpu.make_async_copy(
                rows_a, vout_hbm.at[pl.ds((wid * v_per_w + j) * CHUNK, CHUNK)], sem_a)
            cp_a.start()
            pltpu.async_copy(vemb_hbm.at[vidx_v.at[j + 1]], rows_b, sem_b).wait()
            cp_b = pltpu.make_async_copy(
                rows_b, vout_hbm.at[pl.ds((wid * v_per_w + j + 1) * CHUNK, CHUNK)], sem_b)
            cp_b.start()
            cp_a.wait()
            cp_b.wait()

    return k(sense_emb, v_emb, sidx2, vidx2)


def _tc_loss(sense_rows, v3, b, n_neg, d, blk):
    steps = b // blk

    def body(s_ref, v_ref, o_ref):
        i = pl.program_id(0)
        es = s_ref[...]
        total = jnp.sum(jax.nn.log_sigmoid(jnp.sum(es * v_ref[0], axis=1)))
        for n in range(n_neg):
            q = jnp.sum(es * v_ref[n + 1], axis=1)
            total += jnp.sum(jax.nn.log_sigmoid(-q))

        @pl.when(i == 0)
        def _():
            o_ref[0, 0] = 0.0

        o_ref[0, 0] += -total

    return pl.pallas_call(
        body,
        grid=(steps,),
        in_specs=[
            pl.BlockSpec((blk, d), lambda i: (i, 0)),
            pl.BlockSpec((1 + n_neg, blk, d), lambda i: (0, i, 0)),
        ],
        out_specs=pl.BlockSpec((1, 1), lambda i: (0, 0),
                               memory_space=pltpu.MemorySpace.SMEM),
        out_shape=jax.ShapeDtypeStruct((1, 1), jnp.float32),
    )(sense_rows, v3)


def kernel(pos_u, pos_v, neg_v, rightsense, v_emb, sense_emb):
    b = pos_u.shape[0]
    n_neg = neg_v.shape[1]
    d = v_emb.shape[1]
    k_senses = sense_emb.shape[0] // v_emb.shape[0]

    rs = jnp.asarray(rightsense, dtype=jnp.int32)
    sense_idx = pos_u.astype(jnp.int32) * jnp.int32(k_senses) + rs
    v_idx = jnp.concatenate([pos_v[None, :], neg_v.T], axis=0).reshape(-1)
    n_v = (1 + n_neg) * b

    sidx2 = sense_idx.reshape(b // CHUNK, CHUNK)
    vidx2 = v_idx.reshape(n_v // CHUNK, CHUNK)

    sense_rows, v_rows = _sc_gather(sense_emb, v_emb, sidx2, vidx2, b, n_v)
    v3 = v_rows.reshape(1 + n_neg, b, d)
    out = _tc_loss(sense_rows, v3, b, n_neg, d, blk=2048)
    return out.reshape(())
